# Initial kernel scaffold; baseline (speedup 1.0000x reference)
import jax
import jax.numpy as jnp
from jax.experimental import pallas as pl


def kernel(inputs, emb_w, gate_w, gate_b):
    raise NotImplementedError("write your pallas kernel here")


# speedup vs baseline: 1.3408x; 1.3408x over previous
import functools

import jax
import jax.numpy as jnp
from jax import lax
from jax.experimental import pallas as pl
from jax.experimental.pallas import tpu as pltpu
from jax.experimental.pallas import tpu_sc as plsc

NUM_EMB = 8192
EMB_DIM = 256
COMMIT_SCALE = 1.25

BN = 1024
BK = 1024


def _argmin_body(flat_ref, emb_ref, idx_ref, best_val, best_idx):
    j = pl.program_id(1)
    f = flat_ref[...]
    e = emb_ref[...]
    s = lax.dot_general(e, f, (((1,), (1,)), ((), ())),
                        preferred_element_type=jnp.float32)
    e_norm = jnp.sum(e * e, axis=1, keepdims=True)
    d = e_norm - 2.0 * s
    lmin = jnp.min(d, axis=0, keepdims=True)
    row = lax.broadcasted_iota(jnp.int32, d.shape, 0) + j * BK
    lidx = jnp.min(jnp.where(d == lmin, row, jnp.int32(2**30)),
                   axis=0, keepdims=True)

    @pl.when(j == 0)
    def _():
        best_val[...] = lmin
        best_idx[...] = lidx

    @pl.when(j > 0)
    def _():
        bv = best_val[...]
        take = lmin < bv
        best_val[...] = jnp.where(take, lmin, bv)
        best_idx[...] = jnp.where(take, lidx, best_idx[...])

    @pl.when(j == pl.num_programs(1) - 1)
    def _():
        idx_ref[...] = best_idx[...].reshape(1, 1, BN)


def _argmin_call(flat, emb_w):
    n = flat.shape[0]
    grid = (n // BN, NUM_EMB // BK)
    return pl.pallas_call(
        _argmin_body,
        grid=grid,
        in_specs=[
            pl.BlockSpec((BN, EMB_DIM), lambda i, j: (i, 0)),
            pl.BlockSpec((BK, EMB_DIM), lambda i, j: (j, 0)),
        ],
        out_specs=pl.BlockSpec((1, 1, BN), lambda i, j: (i, 0, 0)),
        out_shape=jax.ShapeDtypeStruct((n // BN, 1, BN), jnp.int32),
        scratch_shapes=[
            pltpu.VMEM((1, BN), jnp.float32),
            pltpu.VMEM((1, BN), jnp.int32),
        ],
        compiler_params=pltpu.CompilerParams(
            dimension_semantics=("parallel", "arbitrary")),
    )(flat, emb_w)


NC, NS = 2, 16
NW = NC * NS


def _gather_call(emb_w, idx):
    b = idx.shape[0]
    b_per_w = b // NW
    mesh = plsc.VectorSubcoreMesh(core_axis_name="c", subcore_axis_name="s",
                                  num_cores=NC, num_subcores=NS)

    @functools.partial(
        pl.kernel, mesh=mesh,
        out_type=jax.ShapeDtypeStruct((b, EMB_DIM), jnp.float32),
        scratch_types=[
            pltpu.VMEM((b_per_w,), jnp.int32),
            pltpu.VMEM((b_per_w, EMB_DIM), jnp.float32),
            pltpu.SemaphoreType.DMA,
        ],
    )
    def gather(table_hbm, idx_hbm, out_hbm, idx_v, rows_v, sem):
        wid = lax.axis_index("s") * NC + lax.axis_index("c")
        base = wid * b_per_w
        pltpu.sync_copy(idx_hbm.at[pl.ds(base, b_per_w)], idx_v)
        pltpu.async_copy(table_hbm.at[idx_v], rows_v, sem).wait()
        pltpu.sync_copy(rows_v, out_hbm.at[pl.ds(base, b_per_w)])

    return gather(emb_w, idx)


BE = 1024


def _epilogue_body(flat_ref, q_ref, gw_ref, gb_ref, out_ref, loss_ref):
    i = pl.program_id(0)
    f = flat_ref[...]
    q = q_ref[...]
    pre = lax.dot_general(f, gw_ref[...], (((1,), (1,)), ((), ())),
                          preferred_element_type=jnp.float32) + gb_ref[...]
    gate = 1.0 / (1.0 + jnp.exp(-pre))
    out_ref[...] = f + q * gate
    diff = q - f
    part = jnp.sum(diff * diff, keepdims=True).reshape(1, 1)

    @pl.when(i == 0)
    def _():
        loss_ref[...] = part

    @pl.when(i > 0)
    def _():
        loss_ref[...] += part


def _epilogue_call(flat, q, gate_w, gate_b):
    n = flat.shape[0]
    grid = (n // BE,)
    return pl.pallas_call(
        _epilogue_body,
        grid=grid,
        in_specs=[
            pl.BlockSpec((BE, EMB_DIM), lambda i: (i, 0)),
            pl.BlockSpec((BE, EMB_DIM), lambda i: (i, 0)),
            pl.BlockSpec((EMB_DIM, EMB_DIM), lambda i: (0, 0)),
            pl.BlockSpec((1, EMB_DIM), lambda i: (0, 0)),
        ],
        out_specs=[
            pl.BlockSpec((BE, EMB_DIM), lambda i: (i, 0)),
            pl.BlockSpec((1, 1), lambda i: (0, 0)),
        ],
        out_shape=[
            jax.ShapeDtypeStruct((n, EMB_DIM), jnp.float32),
            jax.ShapeDtypeStruct((1, 1), jnp.float32),
        ],
        compiler_params=pltpu.CompilerParams(
            dimension_semantics=("arbitrary",)),
    )(flat, q, gate_w, gate_b.reshape(1, EMB_DIM))


def kernel(inputs, emb_w, gate_w, gate_b):
    flat = inputs.reshape(-1, EMB_DIM)
    n = flat.shape[0]
    idx = _argmin_call(flat, emb_w).reshape(n)
    q = _gather_call(emb_w, idx)
    out_flat, loss_sum = _epilogue_call(flat, q, gate_w, gate_b)
    loss = loss_sum[0, 0] * (COMMIT_SCALE / (n * EMB_DIM))
    return (out_flat.reshape(inputs.shape), loss)
